# baseline (speedup 1.0000x reference)
import functools

import jax
import jax.numpy as jnp
from jax import lax
from jax.experimental import pallas as pl
from jax.experimental.pallas import tpu as pltpu
from jax.experimental.pallas import tpu_sc as plsc

BATCH = 16384
EMBED_K = 64
LANES = 16

_info = plsc.get_sparse_core_info()
NC, NS = _info.num_cores, _info.num_subcores
NW = NC * NS
B_PER_W = BATCH // NW
CHUNK = 128
NCHUNK = B_PER_W // CHUNK

_mesh = plsc.VectorSubcoreMesh(core_axis_name="c", subcore_axis_name="s")


@functools.partial(
    pl.kernel,
    out_type=(
        jax.ShapeDtypeStruct((BATCH,), jnp.float32),
        jax.ShapeDtypeStruct((BATCH, EMBED_K), jnp.float32),
        jax.ShapeDtypeStruct((BATCH, EMBED_K), jnp.float32),
    ),
    mesh=_mesh,
    compiler_params=pltpu.CompilerParams(
        needs_layout_passes=False, use_tc_tiling_on_sc=False),
    scratch_types=[
        pltpu.VMEM((NCHUNK, CHUNK), jnp.int32),
        pltpu.VMEM((NCHUNK, CHUNK), jnp.int32),
        pltpu.VMEM((B_PER_W, EMBED_K), jnp.float32),
        pltpu.VMEM((B_PER_W, EMBED_K), jnp.float32),
        pltpu.VMEM((B_PER_W,), jnp.float32),
        pltpu.SemaphoreType.DMA,
        pltpu.SemaphoreType.DMA,
    ],
)
def _bpr_kernel(users_hbm, items_hbm, gu_hbm, gi_hbm,
                xui_hbm, gu_out_hbm, gi_out_hbm,
                idx_u, idx_i, rows_u, rows_i, xui_v, sem_u, sem_i):
    wid = lax.axis_index("s") * NC + lax.axis_index("c")
    base = wid * B_PER_W

    pltpu.sync_copy(users_hbm.at[wid], idx_u)
    pltpu.sync_copy(items_hbm.at[wid], idx_i)

    for j in range(NCHUNK):
        pltpu.async_copy(gu_hbm.at[idx_u.at[j]],
                         rows_u.at[pl.ds(j * CHUNK, CHUNK)], sem_u)
    for j in range(NCHUNK):
        pltpu.async_copy(gi_hbm.at[idx_i.at[j]],
                         rows_i.at[pl.ds(j * CHUNK, CHUNK)], sem_i)
    for j in range(NCHUNK):
        pltpu.make_async_copy(gu_hbm.at[idx_u.at[j]],
                              rows_u.at[pl.ds(j * CHUNK, CHUNK)], sem_u).wait()
    pltpu.sync_copy(rows_u, gu_out_hbm.at[pl.ds(base, B_PER_W)])
    for j in range(NCHUNK):
        pltpu.make_async_copy(gi_hbm.at[idx_i.at[j]],
                              rows_i.at[pl.ds(j * CHUNK, CHUNK)], sem_i).wait()
    pltpu.sync_copy(rows_i, gi_out_hbm.at[pl.ds(base, B_PER_W)])

    lane_iota = jnp.arange(LANES, dtype=jnp.int32)

    def group_body(g, _):
        rbase = g * LANES
        acc = jnp.zeros((LANES,), jnp.float32)
        for rr in range(LANES):
            r = rbase + rr
            s = jnp.zeros((LANES,), jnp.float32)
            for c in range(EMBED_K // LANES):
                u = rows_u[r, pl.ds(c * LANES, LANES)]
                v = rows_i[r, pl.ds(c * LANES, LANES)]
                s = s + u * v
            acc = jnp.where(lane_iota == rr, jnp.sum(s), acc)
        xui_v[pl.ds(rbase, LANES)] = acc
        return 0

    lax.fori_loop(0, B_PER_W // LANES, group_body, 0)
    pltpu.sync_copy(xui_v, xui_hbm.at[pl.ds(base, B_PER_W)])


def kernel(users, items, Gu, Gi):
    users_r = users.reshape(NW, NCHUNK, CHUNK)
    items_r = items.reshape(NW, NCHUNK, CHUNK)
    xui, gamma_u, gamma_i = _bpr_kernel(users_r, items_r, Gu, Gi)
    return (xui, gamma_u, gamma_i)


# speedup vs baseline: 1.5781x; 1.5781x over previous
import functools

import jax
import jax.numpy as jnp
from jax import lax
from jax.experimental import pallas as pl
from jax.experimental.pallas import tpu as pltpu
from jax.experimental.pallas import tpu_sc as plsc

BATCH = 16384
EMBED_K = 64
LANES = 16

_info = plsc.get_sparse_core_info()
NC, NS = _info.num_cores, _info.num_subcores
NW = NC * NS
B_PER_W = BATCH // NW
NPASS = 2
P_ROWS = B_PER_W // NPASS
WINDOW = 64

_mesh = plsc.VectorSubcoreMesh(core_axis_name="c", subcore_axis_name="s")


@functools.partial(
    pl.kernel,
    out_type=(
        jax.ShapeDtypeStruct((BATCH,), jnp.float32),
        jax.ShapeDtypeStruct((BATCH, EMBED_K), jnp.float32),
        jax.ShapeDtypeStruct((BATCH, EMBED_K), jnp.float32),
    ),
    mesh=_mesh,
    compiler_params=pltpu.CompilerParams(needs_layout_passes=False),
    scratch_types=[
        pltpu.VMEM((B_PER_W,), jnp.int32),
        pltpu.VMEM((B_PER_W,), jnp.int32),
        pltpu.VMEM((P_ROWS, EMBED_K), jnp.float32),
        pltpu.VMEM((P_ROWS, EMBED_K), jnp.float32),
        pltpu.VMEM((B_PER_W,), jnp.float32),
        pltpu.SemaphoreType.DMA,
        pltpu.SemaphoreType.DMA,
    ],
)
def _bpr_kernel(users_hbm, items_hbm, gu_hbm, gi_hbm,
                xui_hbm, gu_out_hbm, gi_out_hbm,
                idx_u, idx_i, rows_u, rows_i, xui_v, sem_u, sem_i):
    wid = lax.axis_index("s") * NC + lax.axis_index("c")
    base = wid * B_PER_W

    pltpu.sync_copy(users_hbm.at[pl.ds(base, B_PER_W)], idx_u)
    pltpu.sync_copy(items_hbm.at[pl.ds(base, B_PER_W)], idx_i)

    def drain_one(sem):
        pltpu.make_async_copy(gu_hbm.at[0], rows_u.at[0], sem).wait()

    lane_iota = jnp.arange(LANES, dtype=jnp.int32)
    gwin = WINDOW // LANES

    for p in range(NPASS):
        pbase = p * P_ROWS

        def fetch_group(g, _):
            gb = g * LANES
            vu = idx_u[pl.ds(pbase + gb, LANES)]
            vi = idx_i[pl.ds(pbase + gb, LANES)]
            for rr in range(LANES):
                pltpu.async_copy(gu_hbm.at[vu[rr]], rows_u.at[gb + rr], sem_u)
                pltpu.async_copy(gi_hbm.at[vi[rr]], rows_i.at[gb + rr], sem_i)

            @pl.when(g >= gwin)
            def _():
                for _ in range(LANES):
                    drain_one(sem_u)
                    drain_one(sem_i)

            return 0

        lax.fori_loop(0, P_ROWS // LANES, fetch_group, 0)
        for _ in range(WINDOW):
            drain_one(sem_u)
            drain_one(sem_i)

        pltpu.sync_copy(rows_u, gu_out_hbm.at[pl.ds(base + pbase, P_ROWS)])
        pltpu.sync_copy(rows_i, gi_out_hbm.at[pl.ds(base + pbase, P_ROWS)])

        def group_body(g, _):
            rbase = g * LANES
            acc = jnp.zeros((LANES,), jnp.float32)
            for rr in range(LANES):
                r = rbase + rr
                s = jnp.zeros((LANES,), jnp.float32)
                for c in range(EMBED_K // LANES):
                    u = rows_u[r, pl.ds(c * LANES, LANES)]
                    v = rows_i[r, pl.ds(c * LANES, LANES)]
                    s = s + u * v
                acc = jnp.where(lane_iota == rr, jnp.sum(s), acc)
            xui_v[pl.ds(pbase + rbase, LANES)] = acc
            return 0

        lax.fori_loop(0, P_ROWS // LANES, group_body, 0)

    pltpu.sync_copy(xui_v, xui_hbm.at[pl.ds(base, B_PER_W)])


def kernel(users, items, Gu, Gi):
    return _bpr_kernel(users, items, Gu, Gi)
